# baseline (speedup 1.0000x reference)
import jax
import jax.numpy as jnp
from jax.experimental import pallas as pl
from jax.experimental.pallas import tpu as pltpu

_BN = 16


def _body(idx_ref, w_ref, xlo_ref, xhi_ref, o_ref):
    g = pl.program_id(0)
    th = xlo_ref.shape[-1]
    for j in range(_BN):
        s = idx_ref[g * _BN + j]
        w = w_ref[s].astype(jnp.bfloat16)
        dn = (((1,), (0,)), ((), ()))
        o_ref[j, :, :th] = jax.lax.dot_general(
            w, xlo_ref[j].astype(jnp.bfloat16), dn,
            preferred_element_type=jnp.float32)
        o_ref[j, :, th:] = jax.lax.dot_general(
            w, xhi_ref[j].astype(jnp.bfloat16), dn,
            preferred_element_type=jnp.float32)


def kernel(X, subject_idx, W):
    N, C, T = X.shape
    S = W.shape[0]
    TH = T // 2

    grid_spec = pltpu.PrefetchScalarGridSpec(
        num_scalar_prefetch=1,
        grid=(N // _BN,),
        in_specs=[
            pl.BlockSpec((S, C, C), lambda n, idx: (0, 0, 0)),
            pl.BlockSpec((_BN, C, TH), lambda n, idx: (n, 0, 0)),
            pl.BlockSpec((_BN, C, TH), lambda n, idx: (n, 0, 1)),
        ],
        out_specs=pl.BlockSpec((_BN, C, T), lambda n, idx: (n, 0, 0)),
    )
    return pl.pallas_call(
        _body,
        grid_spec=grid_spec,
        out_shape=jax.ShapeDtypeStruct((N, C, T), jnp.float32),
        compiler_params=pltpu.CompilerParams(
            vmem_limit_bytes=60 * 1024 * 1024,
            dimension_semantics=("arbitrary",),
        ),
    )(subject_idx, W, X, X)


# speedup vs baseline: 1.0063x; 1.0063x over previous
import jax
import jax.numpy as jnp
from jax.experimental import pallas as pl
from jax.experimental.pallas import tpu as pltpu

_BN = 16


def _body(idx_ref, w_ref, x_ref, o_ref):
    g = pl.program_id(0)
    for j in range(_BN):
        s = idx_ref[g * _BN + j]
        w = w_ref[s].astype(jnp.bfloat16)
        x = x_ref[j].astype(jnp.bfloat16)
        o_ref[j] = jax.lax.dot_general(
            w, x,
            dimension_numbers=(((1,), (0,)), ((), ())),
            preferred_element_type=jnp.float32,
        )


def kernel(X, subject_idx, W):
    N, C, T = X.shape
    S = W.shape[0]

    grid_spec = pltpu.PrefetchScalarGridSpec(
        num_scalar_prefetch=1,
        grid=(N // _BN,),
        in_specs=[
            pl.BlockSpec((S, C, C), lambda n, idx: (0, 0, 0)),
            pl.BlockSpec((_BN, C, T), lambda n, idx: (n, 0, 0)),
        ],
        out_specs=pl.BlockSpec((_BN, C, T), lambda n, idx: (n, 0, 0)),
    )
    return pl.pallas_call(
        _body,
        grid_spec=grid_spec,
        out_shape=jax.ShapeDtypeStruct((N, C, T), jnp.float32),
        input_output_aliases={2: 0},
    )(subject_idx, W, X)
